# baseline (speedup 1.0000x reference)
import numpy as np
import jax
import jax.numpy as jnp
from jax.experimental import pallas as pl
from jax.experimental.pallas import tpu as pltpu

N = 128
F_IN = 1433
EMB = 128
L0 = 256
L1 = 128
M = N * (2 * N + 1)

_NEG_INF = float("-inf")


def _cumsum_lane(x):
    s = 1
    while s < N:
        shifted = jnp.concatenate(
            [jnp.zeros((1, s), jnp.float32), x[:, : N - s]], axis=1)
        x = x + shifted
        s *= 2
    return x


def _policy_kernel(adj_ref, fo_ref, embWt_ref, embb_ref, fc1Wt_ref,
                   fc1b_ref, fc2Wt_ref, fc2b_ref, lkWt_ref,
                   lkb_ref, actWt_ref, actb_ref, uni_ref, out_ref, feat_ref,
                   init_ref):
    sub = jax.lax.broadcasted_iota(jnp.int32, (N, 1), 0)
    lane = jax.lax.broadcasted_iota(jnp.int32, (1, N), 1)
    sub2d = jax.lax.broadcasted_iota(jnp.int32, (N, N), 0)
    ident = (jax.lax.broadcasted_iota(jnp.int32, (N, N), 1) == sub2d)
    ident_f = ident.astype(jnp.float32)
    rowmask = (sub != 0).astype(jnp.float32)

    fo = fo_ref[...]
    embWt = embWt_ref[...]
    embb = embb_ref[...]

    fc1Wt = fc1Wt_ref[...]
    fc1b = fc1b_ref[...]

    feat0 = jax.nn.relu(jnp.dot(fo, embWt) + embb)
    feat_ref[...] = feat0
    init_ref[...] = feat0
    out_ref[...] = jnp.zeros((N, EMB), jnp.float32)

    fc2Wt = fc2Wt_ref[...]
    fc2b = fc2b_ref[...]
    lkWt = lkWt_ref[...]
    lkb = lkb_ref[...]
    actWt = actWt_ref[...]
    actb = actb_ref[...]

    def uni_at(p):
        p = jnp.minimum(p, M - 1)
        row = jax.lax.shift_right_logical(p, 7)
        col = jnp.bitwise_and(p, N - 1)
        urow = uni_ref[pl.ds(row, 1), :]
        return jnp.sum(jnp.where(lane == col, urow, 0.0))

    def col_to_lane(x):
        return jnp.sum(jnp.where(ident, jnp.broadcast_to(x, (N, N)), 0.0),
                       axis=0, keepdims=True)

    def row_body(i, ptr):
        alive0 = (adj_ref[pl.ds(i, 1), :] != 0.0).astype(jnp.float32)
        sig0 = jnp.zeros((N, 1), jnp.float32)

        def cond(c):
            return jnp.logical_not(c[3])

        def body(c):
            ptr, alive_f, sig, _ = c
            u = uni_at(ptr)
            u2 = uni_at(ptr + 1)
            msum = jnp.sum(fo * sig, axis=0, keepdims=True)
            sigsum = jnp.sum(sig)
            foi = fo_ref[pl.ds(i, 1), :]

            alive = alive_f > 0.0
            fi = feat_ref[pl.ds(i, 1), :]
            partners = feat_ref[...] * rowmask
            s = jnp.concatenate(
                [jnp.broadcast_to(fi, (N, EMB)), partners], axis=1)
            h1a = jax.nn.relu(
                jnp.dot(s, fc1Wt[:, :EMB]) + fc1b[:, :EMB])
            h1b = jax.nn.relu(
                jnp.dot(s, fc1Wt[:, EMB:]) + fc1b[:, EMB:])
            h2 = jax.nn.relu(
                (jnp.dot(h1a, fc2Wt[:EMB]) + jnp.dot(h1b, fc2Wt[EMB:]))
                + fc2b)
            lkc = jnp.dot(h2, lkWt) + lkb
            lkl = col_to_lane(lkc)

            apl_all = jnp.dot(h2, actWt) + actb
            m2_all = jnp.max(apl_all, axis=1, keepdims=True)
            e2_all = jnp.exp(apl_all - m2_all)
            ap_all = e2_all / jnp.sum(e2_all, axis=1, keepdims=True)
            ap_all = ap_all / jnp.sum(ap_all, axis=1, keepdims=True)
            ap0_all = ap_all[:, 0:1]
            ap1_all = ap_all[:, 1:2]
            c20_all = ap0_all / (ap0_all + ap1_all)

            lk_end = jnp.sum(lkl[:, 0:1])
            m = jnp.maximum(
                jnp.max(jnp.where(alive, lkl, _NEG_INF)), lk_end)
            e = jnp.where(alive, jnp.exp(lkl - m), 0.0)
            e_end = jnp.exp(lk_end - m)
            s1 = jnp.sum(e) + e_end
            p = e / s1
            p_end = e_end / s1
            s2 = jnp.sum(p) + p_end
            p = p / s2
            p_end = p_end / s2
            cdf = _cumsum_lane(p)
            z = jnp.sum(cdf[:, N - 1: N]) + p_end
            cand = jnp.logical_and(alive, (cdf / z) > u)
            raw = jnp.min(jnp.where(cand, lane, 999))
            is_end = raw > N - 1
            node = jnp.minimum(raw, N - 1)

            node_rows = sub2d == node
            nodehot_c = jnp.sum(jnp.where(node_rows, ident_f, 0.0),
                                axis=1, keepdims=True)
            at_all = 1.0 - jnp.maximum(jnp.sign(c20_all - u2), 0.0)
            atf = jnp.sum(at_all * nodehot_c)

            upd = jnp.logical_not(is_end)

            @pl.when(upd)
            def _():
                fon = fo_ref[pl.ds(node, 1), :]
                cnt = sigsum + atf + 1.0
                hv_in = ((msum + atf * fon) + foi) / cnt
                hv = jax.nn.relu(jnp.dot(hv_in, embWt) + embb)
                hut = init_ref[pl.ds(node, 1), :]
                feat_ref[pl.ds(i, 1), :] = hv
                feat_ref[pl.ds(node, 1), :] = hut
                out_ref[pl.ds(i, 1), :] = hv
                out_ref[pl.ds(node, 1), :] = hut

            updf = jnp.where(upd, 1.0, 0.0)
            nodehot_l = (lane == node).astype(jnp.float32)
            alive_f = alive_f * (1.0 - nodehot_l * updf)
            sig = jnp.maximum(sig, nodehot_c * (atf * updf))
            ptr = ptr + jnp.where(is_end, 1, 2)
            return (ptr, alive_f, sig, is_end)

        carry = jax.lax.while_loop(
            cond, body, (ptr, alive0, sig0, jnp.bool_(False)))
        return carry[0]

    jax.lax.fori_loop(0, N, row_body, jnp.int32(0))


def kernel(adj, feature_origin, labels, emb_W, emb_b, fc1_W, fc1_b, fc2_W,
           fc2_b, lk_W, lk_b, act_W, act_b, rt1_W, rt1_b, rt2_W, rt2_b):
    del labels, rt1_W, rt1_b, rt2_W, rt2_b
    rng = np.random.default_rng(0)
    uniforms = jnp.asarray(rng.random(M), dtype=jnp.float32).reshape(
        M // N, N)

    out = pl.pallas_call(
        _policy_kernel,
        out_shape=jax.ShapeDtypeStruct((N, EMB), jnp.float32),
        scratch_shapes=[pltpu.VMEM((N, EMB), jnp.float32),
                        pltpu.VMEM((N, EMB), jnp.float32)],
    )(
        adj,
        feature_origin,
        emb_W.T,
        emb_b.reshape(1, EMB),
        fc1_W.T,
        fc1_b.reshape(1, L0),
        fc2_W.T,
        fc2_b.reshape(1, L1),
        lk_W.T,
        lk_b.reshape(1, 1),
        act_W.T,
        act_b.reshape(1, 2),
        uniforms,
    )
    return out


# speedup vs baseline: 1.0196x; 1.0196x over previous
import numpy as np
import jax
import jax.numpy as jnp
from jax.experimental import pallas as pl
from jax.experimental.pallas import tpu as pltpu

N = 128
F_IN = 1433
EMB = 128
L0 = 256
L1 = 128
M = N * (2 * N + 1)

_NEG_INF = float("-inf")


def _cumsum_lane(x):
    s = 1
    while s < N:
        shifted = jnp.concatenate(
            [jnp.zeros((1, s), jnp.float32), x[:, : N - s]], axis=1)
        x = x + shifted
        s *= 2
    return x


def _policy_kernel(adj_ref, fo_ref, embWt_ref, embb_ref, fc1Wt_ref,
                   fc1b_ref, fc2Wt_ref, fc2b_ref, lkWt_ref,
                   lkb_ref, actWt_ref, actb_ref, uni_ref, out_ref, feat_ref,
                   init_ref):
    sub = jax.lax.broadcasted_iota(jnp.int32, (N, 1), 0)
    lane = jax.lax.broadcasted_iota(jnp.int32, (1, N), 1)
    sub2d = jax.lax.broadcasted_iota(jnp.int32, (N, N), 0)
    ident = (jax.lax.broadcasted_iota(jnp.int32, (N, N), 1) == sub2d)
    ident_f = ident.astype(jnp.float32)
    rowmask = (sub != 0).astype(jnp.float32)

    fo = fo_ref[...]
    embWt = embWt_ref[...]
    embb = embb_ref[...]

    fc1Wt = fc1Wt_ref[...]
    fc1b = fc1b_ref[...]

    feat0 = jax.nn.relu(jnp.dot(fo, embWt) + embb)
    feat_ref[...] = feat0
    init_ref[...] = feat0
    out_ref[...] = jnp.zeros((N, EMB), jnp.float32)

    fc2Wt = fc2Wt_ref[...]
    fc2b = fc2b_ref[...]
    lkWt = lkWt_ref[...]
    lkb = lkb_ref[...]
    actWt = actWt_ref[...]
    actb = actb_ref[...]

    def uni_at(p):
        p = jnp.minimum(p, M - 1)
        row = jax.lax.shift_right_logical(p, 7)
        col = jnp.bitwise_and(p, N - 1)
        urow = uni_ref[pl.ds(row, 1), :]
        return jnp.sum(jnp.where(lane == col, urow, 0.0))

    def col_to_lane(x):
        return jnp.sum(jnp.where(ident, jnp.broadcast_to(x, (N, N)), 0.0),
                       axis=0, keepdims=True)

    def row_body(i, ptr):
        alive0 = (adj_ref[pl.ds(i, 1), :] != 0.0).astype(jnp.float32)
        sig0 = jnp.zeros((N, 1), jnp.float32)

        def cond(c):
            return jnp.logical_not(c[3])

        def body(c):
            ptr, alive_f, sig, _ = c
            u = uni_at(ptr)
            u2 = uni_at(ptr + 1)
            msum = jnp.sum(fo * sig, axis=0, keepdims=True)
            sigsum = jnp.sum(sig)
            foi = fo_ref[pl.ds(i, 1), :]

            alive = alive_f > 0.0
            fi = feat_ref[pl.ds(i, 1), :]
            partners = feat_ref[...] * rowmask
            s = jnp.concatenate(
                [jnp.broadcast_to(fi, (N, EMB)), partners], axis=1)
            h1 = jax.nn.relu(jnp.dot(s, fc1Wt) + fc1b)
            h2 = jax.nn.relu(jnp.dot(h1, fc2Wt) + fc2b)
            lkc = jnp.dot(h2, lkWt) + lkb
            lkl = col_to_lane(lkc)

            apl_all = jnp.dot(h2, actWt) + actb
            m2_all = jnp.max(apl_all, axis=1, keepdims=True)
            e2_all = jnp.exp(apl_all - m2_all)
            ap_all = e2_all / jnp.sum(e2_all, axis=1, keepdims=True)
            ap_all = ap_all / jnp.sum(ap_all, axis=1, keepdims=True)
            ap0_all = ap_all[:, 0:1]
            ap1_all = ap_all[:, 1:2]
            c20_all = ap0_all / (ap0_all + ap1_all)

            lk_end = jnp.sum(lkl[:, 0:1])
            m = jnp.maximum(
                jnp.max(jnp.where(alive, lkl, _NEG_INF)), lk_end)
            e = jnp.where(alive, jnp.exp(lkl - m), 0.0)
            e_end = jnp.exp(lk_end - m)
            s1 = jnp.sum(e) + e_end
            p = e / s1
            p_end = e_end / s1
            s2 = jnp.sum(p) + p_end
            p = p / s2
            p_end = p_end / s2
            cdf = _cumsum_lane(p)
            z = jnp.sum(cdf[:, N - 1: N]) + p_end
            cand = jnp.logical_and(alive, (cdf / z) > u)
            raw = jnp.min(jnp.where(cand, lane, 999))
            is_end = raw > N - 1
            node = jnp.minimum(raw, N - 1)

            node_rows = sub2d == node
            nodehot_c = jnp.sum(jnp.where(node_rows, ident_f, 0.0),
                                axis=1, keepdims=True)
            at_all = 1.0 - jnp.maximum(jnp.sign(c20_all - u2), 0.0)
            atf = jnp.sum(at_all * nodehot_c)

            upd = jnp.logical_not(is_end)

            @pl.when(upd)
            def _():
                fon = fo_ref[pl.ds(node, 1), :]
                cnt = sigsum + atf + 1.0
                hv_in = ((msum + atf * fon) + foi) / cnt
                hv = jax.nn.relu(jnp.dot(hv_in, embWt) + embb)
                hut = init_ref[pl.ds(node, 1), :]
                feat_ref[pl.ds(i, 1), :] = hv
                feat_ref[pl.ds(node, 1), :] = hut
                out_ref[pl.ds(i, 1), :] = hv
                out_ref[pl.ds(node, 1), :] = hut

            updf = jnp.where(upd, 1.0, 0.0)
            nodehot_l = (lane == node).astype(jnp.float32)
            alive_f = alive_f * (1.0 - nodehot_l * updf)
            sig = jnp.maximum(sig, nodehot_c * (atf * updf))
            ptr = ptr + jnp.where(is_end, 1, 2)
            return (ptr, alive_f, sig, is_end)

        carry = jax.lax.while_loop(
            cond, body, (ptr, alive0, sig0, jnp.bool_(False)))
        return carry[0]

    jax.lax.fori_loop(0, N, row_body, jnp.int32(0))


def kernel(adj, feature_origin, labels, emb_W, emb_b, fc1_W, fc1_b, fc2_W,
           fc2_b, lk_W, lk_b, act_W, act_b, rt1_W, rt1_b, rt2_W, rt2_b):
    del labels, rt1_W, rt1_b, rt2_W, rt2_b
    rng = np.random.default_rng(0)
    uniforms = jnp.asarray(rng.random(M), dtype=jnp.float32).reshape(
        M // N, N)

    out = pl.pallas_call(
        _policy_kernel,
        out_shape=jax.ShapeDtypeStruct((N, EMB), jnp.float32),
        scratch_shapes=[pltpu.VMEM((N, EMB), jnp.float32),
                        pltpu.VMEM((N, EMB), jnp.float32)],
    )(
        adj,
        feature_origin,
        emb_W.T,
        emb_b.reshape(1, EMB),
        fc1_W.T,
        fc1_b.reshape(1, L0),
        fc2_W.T,
        fc2_b.reshape(1, L1),
        lk_W.T,
        lk_b.reshape(1, 1),
        act_W.T,
        act_b.reshape(1, 2),
        uniforms,
    )
    return out
